# Initial kernel scaffold; baseline (speedup 1.0000x reference)
import jax
import jax.numpy as jnp
from jax.experimental import pallas as pl


def kernel(img):
    raise NotImplementedError("write your pallas kernel here")


# speedup vs baseline: 1.5737x; 1.5737x over previous
import functools

import numpy as np
import jax
import jax.numpy as jnp
from jax import lax
from jax.experimental import pallas as pl
from jax.experimental.pallas import tpu as pltpu
from jax.experimental.pallas import tpu_sc as plsc

_MIN_DISABLED = 2
_MAX_DISABLED = 8
_NEIGHBOR_PROB = 0.5


def _disabled_tofs(tof_count):
    rng = np.random.default_rng(0)
    disabled_count = int(rng.integers(_MIN_DISABLED, _MAX_DISABLED + 1))
    initial = int(rng.integers(0, tof_count))
    disabled = [initial]
    tof_list = [int(t) for t in rng.permutation(tof_count) if int(t) != initial]
    for _ in range(disabled_count - 1):
        rv = float(rng.random())
        perm = rng.permutation(len(disabled))
        permuted = [disabled[int(j)] for j in perm]
        if rv < _NEIGHBOR_PROB:
            if rv < _NEIGHBOR_PROB / 2:
                for cur in permuted:
                    new_neighbor = (cur + 1) % tof_count
                    if new_neighbor not in disabled:
                        disabled.append(new_neighbor)
                        tof_list = [t for t in tof_list if t != new_neighbor]
                        break
            else:
                opposite_found = False
                for cur in permuted:
                    new_opposite = (cur + tof_count // 2) % tof_count
                    if new_opposite not in disabled:
                        disabled.append(new_opposite)
                        tof_list = [t for t in tof_list if t != new_opposite]
                        opposite_found = True
                        break
                if not opposite_found:
                    new_element = tof_list[0]
                    tof_list = [t for t in tof_list if t != new_element]
                    disabled.append(new_element)
        else:
            new_element = tof_list[0]
            tof_list = [t for t in tof_list if t != new_element]
            disabled.append(new_element)
    return sorted(set(int(d) for d in disabled))


_ROWS, _COLS = 65536, 512
_DISABLED = _disabled_tofs(_COLS)

_NC, _NS, _L = 2, 16, 16
_NW = _NC * _NS
_RPW = _ROWS // _NW
_R = 64
_NCHUNK = _RPW // _R


def _body(img_hbm, out_hbm, buf, _):
    wid = lax.axis_index("s") * _NC + lax.axis_index("c")
    base = wid * _RPW

    lane = lax.iota(jnp.int32, _L)
    groups = []
    for g0 in sorted({(c // _L) * _L for c in _DISABLED}):
        cond = None
        for c in _DISABLED:
            if c // _L == g0 // _L:
                eq = lane == (c - g0)
                cond = eq if cond is None else (cond | eq)
        groups.append((g0, cond))

    def chunk(i, carry):
        r0 = base + i * _R
        pltpu.sync_copy(img_hbm.at[pl.ds(r0, _R)], buf)

        def zero_row(r, c2):
            for g0, cond in groups:
                v = buf[r, pl.ds(g0, _L)]
                buf[r, pl.ds(g0, _L)] = jnp.where(cond, 0.0, v)
            return c2

        lax.fori_loop(0, _R, zero_row, 0)
        pltpu.sync_copy(buf, out_hbm.at[pl.ds(r0, _R)])
        return carry

    lax.fori_loop(0, _NCHUNK, chunk, 0)


def kernel(img):
    mesh = plsc.VectorSubcoreMesh(
        core_axis_name="c", subcore_axis_name="s",
        num_cores=_NC, num_subcores=_NS,
    )
    run = pl.kernel(
        _body,
        out_type=jax.ShapeDtypeStruct((_ROWS, _COLS), jnp.float32),
        mesh=mesh,
        scratch_types=[
            pltpu.VMEM((_R, _COLS), jnp.float32),
            pltpu.SemaphoreType.DMA,
        ],
    )
    return run(img)


# speedup vs baseline: 1.8440x; 1.1718x over previous
import functools

import numpy as np
import jax
import jax.numpy as jnp
from jax import lax
from jax.experimental import pallas as pl
from jax.experimental.pallas import tpu as pltpu
from jax.experimental.pallas import tpu_sc as plsc

_MIN_DISABLED = 2
_MAX_DISABLED = 8
_NEIGHBOR_PROB = 0.5


def _disabled_tofs(tof_count):
    rng = np.random.default_rng(0)
    disabled_count = int(rng.integers(_MIN_DISABLED, _MAX_DISABLED + 1))
    initial = int(rng.integers(0, tof_count))
    disabled = [initial]
    tof_list = [int(t) for t in rng.permutation(tof_count) if int(t) != initial]
    for _ in range(disabled_count - 1):
        rv = float(rng.random())
        perm = rng.permutation(len(disabled))
        permuted = [disabled[int(j)] for j in perm]
        if rv < _NEIGHBOR_PROB:
            if rv < _NEIGHBOR_PROB / 2:
                for cur in permuted:
                    new_neighbor = (cur + 1) % tof_count
                    if new_neighbor not in disabled:
                        disabled.append(new_neighbor)
                        tof_list = [t for t in tof_list if t != new_neighbor]
                        break
            else:
                opposite_found = False
                for cur in permuted:
                    new_opposite = (cur + tof_count // 2) % tof_count
                    if new_opposite not in disabled:
                        disabled.append(new_opposite)
                        tof_list = [t for t in tof_list if t != new_opposite]
                        opposite_found = True
                        break
                if not opposite_found:
                    new_element = tof_list[0]
                    tof_list = [t for t in tof_list if t != new_element]
                    disabled.append(new_element)
        else:
            new_element = tof_list[0]
            tof_list = [t for t in tof_list if t != new_element]
            disabled.append(new_element)
    return sorted(set(int(d) for d in disabled))


_ROWS, _COLS = 65536, 512
_DISABLED = _disabled_tofs(_COLS)

_NC, _NS, _L = 2, 16, 16
_NW = _NC * _NS
_RPW = _ROWS // _NW
_R = 64
_NCHUNK = _RPW // _R


def _body(img_hbm, out_hbm, buf0, buf1, sin0, sin1, sout0, sout1):
    wid = lax.axis_index("s") * _NC + lax.axis_index("c")
    base = wid * _RPW

    lane = lax.iota(jnp.int32, _L)
    groups = []
    for g0 in sorted({(c // _L) * _L for c in _DISABLED}):
        cond = None
        for c in _DISABLED:
            if c // _L == g0 // _L:
                eq = lane == (c - g0)
                cond = eq if cond is None else (cond | eq)
        groups.append((g0, cond))

    bufs = (buf0, buf1)
    sins = (sin0, sin1)
    souts = (sout0, sout1)

    def in_copy(i):
        return pltpu.make_async_copy(
            img_hbm.at[pl.ds(base + i * _R, _R)], bufs[i % 2], sins[i % 2])

    def out_copy(i):
        return pltpu.make_async_copy(
            bufs[i % 2], out_hbm.at[pl.ds(base + i * _R, _R)], souts[i % 2])

    def blend(buf):
        def zero_row(r, carry):
            for g0, cond in groups:
                v = buf[r, pl.ds(g0, _L)]
                buf[r, pl.ds(g0, _L)] = jnp.where(cond, 0.0, v)
            return carry
        lax.fori_loop(0, _R, zero_row, 0)

    in_copy(0).start()
    for i in range(_NCHUNK):
        in_copy(i).wait()
        blend(bufs[i % 2])
        out_copy(i).start()
        if i + 1 < _NCHUNK:
            if i >= 1:
                out_copy(i - 1).wait()
            in_copy(i + 1).start()
    out_copy(_NCHUNK - 1).wait()
    if _NCHUNK >= 2:
        out_copy(_NCHUNK - 2).wait()


def kernel(img):
    mesh = plsc.VectorSubcoreMesh(
        core_axis_name="c", subcore_axis_name="s",
        num_cores=_NC, num_subcores=_NS,
    )
    run = pl.kernel(
        _body,
        out_type=jax.ShapeDtypeStruct((_ROWS, _COLS), jnp.float32),
        mesh=mesh,
        scratch_types=[
            pltpu.VMEM((_R, _COLS), jnp.float32),
            pltpu.VMEM((_R, _COLS), jnp.float32),
            pltpu.SemaphoreType.DMA,
            pltpu.SemaphoreType.DMA,
            pltpu.SemaphoreType.DMA,
            pltpu.SemaphoreType.DMA,
        ],
    )
    return run(img)
